# baseline (speedup 1.0000x reference)
import dataclasses
import functools

import jax
import jax.numpy as jnp
from jax.experimental import pallas as pl
from jax.experimental.pallas import tpu as pltpu
from jax.experimental.pallas import tpu_sc as plsc

_K = 20
_KPAD = 32
_R = 256

_INTERPRET = False


def _topk_body(x_ref, xt_ref, out_ref, dist_ref):
    b = pl.program_id(0)
    t = pl.program_id(1)
    n = x_ref.shape[2]

    x = x_ref[0]
    xt = xt_ref[0, pl.ds(t * _R, _R), :]
    x0, x1, x2 = x[0:1, :], x[1:2, :], x[2:3, :]
    a0, a1, a2 = xt[:, 0:1], xt[:, 1:2], xt[:, 2:3]
    xx = (x0 * x0 + x1 * x1) + x2 * x2
    rr = (a0 * a0 + a1 * a1) + a2 * a2
    s = jax.lax.dot_general(
        xt.astype(jnp.bfloat16), x.astype(jnp.bfloat16),
        (((1,), (0,)), ((), ())),
        preferred_element_type=jnp.float32,
    )
    inner = -2.0 * s
    dist_ref[...] = ((-xx) - inner) - rr

    col = jax.lax.broadcasted_iota(jnp.int32, (_R, n), 1)
    lane_k = jax.lax.broadcasted_iota(jnp.int32, (_R, _KPAD), 1)
    neg_inf = jnp.float32(-jnp.inf)

    m0 = jnp.max(dist_ref[...], axis=1, keepdims=True)

    def body(k, carry):
        m, idx = carry
        d = dist_ref[...]
        cand = jnp.where(d == m, col, n)
        j = jnp.min(cand, axis=1, keepdims=True)
        idx = jnp.where(lane_k == k, j, idx)
        dnew = jnp.where(col == j, neg_inf, d)
        dist_ref[...] = dnew
        m = jnp.max(dnew, axis=1, keepdims=True)
        return m, idx

    _, idx = jax.lax.fori_loop(
        0, _K, body, (m0, jnp.zeros((_R, _KPAD), jnp.int32)))
    del b
    out_ref[0] = idx


def _topk_indices(coor):
    b, c, n = coor.shape
    xt = jnp.transpose(coor, (0, 2, 1))
    return pl.pallas_call(
        _topk_body,
        grid=(b, n // _R),
        in_specs=[
            pl.BlockSpec((1, c, n), lambda bb, tt: (bb, 0, 0)),
            pl.BlockSpec((1, n, c), lambda bb, tt: (bb, 0, 0)),
        ],
        out_specs=pl.BlockSpec((1, _R, _KPAD), lambda bb, tt: (bb, tt, 0)),
        out_shape=jax.ShapeDtypeStruct((b, n, _KPAD), jnp.int32),
        scratch_shapes=[pltpu.VMEM((_R, n), jnp.float32)],
        interpret=_INTERPRET,
    )(coor, xt)


_NW = 32
_CHUNK = 2048
_NCH = 6


def _sc_gather(table, idx_flat):
    m = idx_flat.shape[0]
    per_w = m // _NW
    w_per_b = m // table.shape[0] // per_w
    tab_w = table.shape[1]
    mesh = plsc.VectorSubcoreMesh(core_axis_name="c", subcore_axis_name="s")
    cp = pltpu.CompilerParams()
    if "needs_layout_passes" in pltpu.CompilerParams.__dataclass_fields__:
        cp = dataclasses.replace(cp, needs_layout_passes=False)

    @functools.partial(
        pl.kernel,
        out_type=jax.ShapeDtypeStruct((_NCH, m), jnp.float32),
        mesh=mesh,
        compiler_params=cp,
        scratch_types=[
            pltpu.VMEM((tab_w,), jnp.float32),
            pltpu.VMEM((_CHUNK,), jnp.int32),
            pltpu.VMEM((_NCH, _CHUNK), jnp.float32),
        ],
    )
    def gather_kernel(tab_hbm, i_hbm, o_hbm, tab_v, idx_v, out_v):
        wid = jax.lax.axis_index("s") * 2 + jax.lax.axis_index("c")
        base = wid * per_w
        pltpu.sync_copy(tab_hbm.at[wid // w_per_b], tab_v)

        @pl.loop(0, per_w, step=_CHUNK)
        def _(off):
            pltpu.sync_copy(i_hbm.at[pl.ds(base + off, _CHUNK)], idx_v)

            @pl.loop(0, _CHUNK, step=16)
            def _(g):
                fl = idx_v[pl.ds(g, 16)] * _NCH
                for ch in range(_NCH):
                    out_v[ch, pl.ds(g, 16)] = plsc.load_gather(
                        tab_v, [fl + ch])

            pltpu.sync_copy(out_v, o_hbm.at[:, pl.ds(base + off, _CHUNK)])

    return gather_kernel(table, idx_flat)


def kernel(coor, nor):
    b, c, n = coor.shape
    idx = _topk_indices(coor)
    idx_flat = idx[:, :, :_K].reshape(b * n * _K)

    feat = jnp.concatenate([coor, nor], axis=1)
    table = jnp.transpose(feat, (0, 2, 1)).reshape(b, n * 2 * c)

    if _INTERPRET:
        tab2 = table.reshape(b, n, 2 * c)
        g = jnp.take_along_axis(
            tab2, idx[:, :, :_K].reshape(b, n * _K)[:, :, None], axis=1
        )
        g = jnp.transpose(g.reshape(b * n * _K, 2 * c), (1, 0))
    else:
        g = _sc_gather(table, idx_flat)

    nb = jnp.transpose(g.reshape(2 * c, b, n, _K), (1, 0, 2, 3))
    ctr = jnp.broadcast_to(feat[:, :, :, None], (b, 2 * c, n, _K))
    return jnp.concatenate(
        [nb[:, 0:c], ctr[:, 0:c], nb[:, c:], ctr[:, c:]], axis=1
    )


# speedup vs baseline: 1.0706x; 1.0706x over previous
import dataclasses
import functools

import jax
import jax.numpy as jnp
from jax.experimental import pallas as pl
from jax.experimental.pallas import tpu as pltpu
from jax.experimental.pallas import tpu_sc as plsc

_K = 20
_KPAD = 32
_R = 512

_INTERPRET = False


def _topk_body(x_ref, xt_ref, out_ref, dist_ref):
    b = pl.program_id(0)
    t = pl.program_id(1)
    n = x_ref.shape[2]

    x = x_ref[0]
    xt = xt_ref[0, pl.ds(t * _R, _R), :]
    x0, x1, x2 = x[0:1, :], x[1:2, :], x[2:3, :]
    a0, a1, a2 = xt[:, 0:1], xt[:, 1:2], xt[:, 2:3]
    xx = (x0 * x0 + x1 * x1) + x2 * x2
    rr = (a0 * a0 + a1 * a1) + a2 * a2
    s = jax.lax.dot_general(
        xt.astype(jnp.bfloat16), x.astype(jnp.bfloat16),
        (((1,), (0,)), ((), ())),
        preferred_element_type=jnp.float32,
    )
    inner = -2.0 * s
    dist_ref[...] = ((-xx) - inner) - rr

    col = jax.lax.broadcasted_iota(jnp.int32, (_R, n), 1)
    lane_k = jax.lax.broadcasted_iota(jnp.int32, (_R, _KPAD), 1)
    neg_inf = jnp.float32(-jnp.inf)

    m0 = jnp.max(dist_ref[...], axis=1, keepdims=True)

    def body(k, carry):
        m, idx = carry
        d = dist_ref[...]
        cand = jnp.where(d == m, col, n)
        j = jnp.min(cand, axis=1, keepdims=True)
        idx = jnp.where(lane_k == k, j, idx)
        dnew = jnp.where(col == j, neg_inf, d)
        dist_ref[...] = dnew
        m = jnp.max(dnew, axis=1, keepdims=True)
        return m, idx

    _, idx = jax.lax.fori_loop(
        0, _K, body, (m0, jnp.zeros((_R, _KPAD), jnp.int32)))
    del b
    out_ref[0] = idx


def _topk_indices(coor):
    b, c, n = coor.shape
    xt = jnp.transpose(coor, (0, 2, 1))
    return pl.pallas_call(
        _topk_body,
        grid=(b, n // _R),
        in_specs=[
            pl.BlockSpec((1, c, n), lambda bb, tt: (bb, 0, 0)),
            pl.BlockSpec((1, n, c), lambda bb, tt: (bb, 0, 0)),
        ],
        out_specs=pl.BlockSpec((1, _R, _KPAD), lambda bb, tt: (bb, tt, 0)),
        out_shape=jax.ShapeDtypeStruct((b, n, _KPAD), jnp.int32),
        scratch_shapes=[pltpu.VMEM((_R, n), jnp.float32)],
        interpret=_INTERPRET,
    )(coor, xt)


_NW = 32
_CHUNK = 2048
_NCH = 6


def _sc_gather(table, idx_flat):
    m = idx_flat.shape[0]
    per_w = m // _NW
    w_per_b = m // table.shape[0] // per_w
    tab_w = table.shape[1]
    mesh = plsc.VectorSubcoreMesh(core_axis_name="c", subcore_axis_name="s")
    cp = pltpu.CompilerParams()
    if "needs_layout_passes" in pltpu.CompilerParams.__dataclass_fields__:
        cp = dataclasses.replace(cp, needs_layout_passes=False)

    @functools.partial(
        pl.kernel,
        out_type=jax.ShapeDtypeStruct((_NCH, m), jnp.float32),
        mesh=mesh,
        compiler_params=cp,
        scratch_types=[
            pltpu.VMEM((tab_w,), jnp.float32),
            pltpu.VMEM((_CHUNK,), jnp.int32),
            pltpu.VMEM((_NCH, _CHUNK), jnp.float32),
        ],
    )
    def gather_kernel(tab_hbm, i_hbm, o_hbm, tab_v, idx_v, out_v):
        wid = jax.lax.axis_index("s") * 2 + jax.lax.axis_index("c")
        base = wid * per_w
        pltpu.sync_copy(tab_hbm.at[wid // w_per_b], tab_v)

        @pl.loop(0, per_w, step=_CHUNK)
        def _(off):
            pltpu.sync_copy(i_hbm.at[pl.ds(base + off, _CHUNK)], idx_v)

            @pl.loop(0, _CHUNK, step=16)
            def _(g):
                fl = idx_v[pl.ds(g, 16)] * _NCH
                for ch in range(_NCH):
                    out_v[ch, pl.ds(g, 16)] = plsc.load_gather(
                        tab_v, [fl + ch])

            pltpu.sync_copy(out_v, o_hbm.at[:, pl.ds(base + off, _CHUNK)])

    return gather_kernel(table, idx_flat)


def kernel(coor, nor):
    b, c, n = coor.shape
    idx = _topk_indices(coor)
    idx_flat = idx[:, :, :_K].reshape(b * n * _K)

    feat = jnp.concatenate([coor, nor], axis=1)
    table = jnp.transpose(feat, (0, 2, 1)).reshape(b, n * 2 * c)

    if _INTERPRET:
        tab2 = table.reshape(b, n, 2 * c)
        g = jnp.take_along_axis(
            tab2, idx[:, :, :_K].reshape(b, n * _K)[:, :, None], axis=1
        )
        g = jnp.transpose(g.reshape(b * n * _K, 2 * c), (1, 0))
    else:
        g = _sc_gather(table, idx_flat)

    nb = jnp.transpose(g.reshape(2 * c, b, n, _K), (1, 0, 2, 3))
    ctr = jnp.broadcast_to(feat[:, :, :, None], (b, 2 * c, n, _K))
    return jnp.concatenate(
        [nb[:, 0:c], ctr[:, 0:c], nb[:, c:], ctr[:, c:]], axis=1
    )


# speedup vs baseline: 1.1610x; 1.0845x over previous
import dataclasses
import functools

import jax
import jax.numpy as jnp
from jax.experimental import pallas as pl
from jax.experimental.pallas import tpu as pltpu
from jax.experimental.pallas import tpu_sc as plsc

_K = 20
_KPAD = 32
_R = 512

_INTERPRET = False


def _topk_body(x_ref, xt_ref, out_ref, dist_ref):
    b = pl.program_id(0)
    t = pl.program_id(1)
    n = x_ref.shape[2]

    x = x_ref[0]
    xt = xt_ref[0, pl.ds(t * _R, _R), :]
    x0, x1, x2 = x[0:1, :], x[1:2, :], x[2:3, :]
    a0, a1, a2 = xt[:, 0:1], xt[:, 1:2], xt[:, 2:3]
    xx = (x0 * x0 + x1 * x1) + x2 * x2
    rr = (a0 * a0 + a1 * a1) + a2 * a2
    s = jax.lax.dot_general(
        xt.astype(jnp.bfloat16), x.astype(jnp.bfloat16),
        (((1,), (0,)), ((), ())),
        preferred_element_type=jnp.float32,
    )
    inner = -2.0 * s
    dist_ref[...] = ((-xx) - inner) - rr

    col = jax.lax.broadcasted_iota(jnp.int32, (_R, n), 1)
    lane_k = jax.lax.broadcasted_iota(jnp.int32, (_R, _KPAD), 1)
    neg_inf = jnp.float32(-jnp.inf)

    def body(k, idx):
        d = dist_ref[...]
        m = jnp.max(d, axis=1, keepdims=True)
        cand = jnp.where(d == m, col, n)
        j = jnp.min(cand, axis=1, keepdims=True)
        dist_ref[...] = jnp.where(col == j, neg_inf, d)
        return jnp.where(lane_k == k, j, idx)

    idx = jax.lax.fori_loop(0, _K, body, jnp.zeros((_R, _KPAD), jnp.int32))
    del b
    out_ref[0] = idx


def _topk_indices(coor):
    b, c, n = coor.shape
    xt = jnp.transpose(coor, (0, 2, 1))
    return pl.pallas_call(
        _topk_body,
        grid=(b, n // _R),
        in_specs=[
            pl.BlockSpec((1, c, n), lambda bb, tt: (bb, 0, 0)),
            pl.BlockSpec((1, n, c), lambda bb, tt: (bb, 0, 0)),
        ],
        out_specs=pl.BlockSpec((1, _R, _KPAD), lambda bb, tt: (bb, tt, 0)),
        out_shape=jax.ShapeDtypeStruct((b, n, _KPAD), jnp.int32),
        scratch_shapes=[pltpu.VMEM((_R, n), jnp.float32)],
        interpret=_INTERPRET,
    )(coor, xt)


_NW = 32
_CHUNK = 2048
_NCH = 6


def _sc_gather(table, idx_flat):
    m = idx_flat.shape[0]
    per_w = m // _NW
    w_per_b = m // table.shape[0] // per_w
    tab_w = table.shape[1]
    mesh = plsc.VectorSubcoreMesh(core_axis_name="c", subcore_axis_name="s")
    cp = pltpu.CompilerParams()
    if "needs_layout_passes" in pltpu.CompilerParams.__dataclass_fields__:
        cp = dataclasses.replace(cp, needs_layout_passes=False)

    @functools.partial(
        pl.kernel,
        out_type=jax.ShapeDtypeStruct((_NCH, m), jnp.float32),
        mesh=mesh,
        compiler_params=cp,
        scratch_types=[
            pltpu.VMEM((tab_w,), jnp.float32),
            pltpu.VMEM((_CHUNK,), jnp.int32),
            pltpu.VMEM((_NCH, _CHUNK), jnp.float32),
        ],
    )
    def gather_kernel(tab_hbm, i_hbm, o_hbm, tab_v, idx_v, out_v):
        wid = jax.lax.axis_index("s") * 2 + jax.lax.axis_index("c")
        base = wid * per_w
        pltpu.sync_copy(tab_hbm.at[wid // w_per_b], tab_v)

        @pl.loop(0, per_w, step=_CHUNK)
        def _(off):
            pltpu.sync_copy(i_hbm.at[pl.ds(base + off, _CHUNK)], idx_v)

            @pl.loop(0, _CHUNK, step=16)
            def _(g):
                fl = idx_v[pl.ds(g, 16)] * _NCH
                for ch in range(_NCH):
                    out_v[ch, pl.ds(g, 16)] = plsc.load_gather(
                        tab_v, [fl + ch])

            pltpu.sync_copy(out_v, o_hbm.at[:, pl.ds(base + off, _CHUNK)])

    return gather_kernel(table, idx_flat)


def kernel(coor, nor):
    b, c, n = coor.shape
    idx = _topk_indices(coor)
    idx_flat = idx[:, :, :_K].reshape(b * n * _K)

    feat = jnp.concatenate([coor, nor], axis=1)
    table = jnp.transpose(feat, (0, 2, 1)).reshape(b, n * 2 * c)

    if _INTERPRET:
        tab2 = table.reshape(b, n, 2 * c)
        g = jnp.take_along_axis(
            tab2, idx[:, :, :_K].reshape(b, n * _K)[:, :, None], axis=1
        )
        g = jnp.transpose(g.reshape(b * n * _K, 2 * c), (1, 0))
    else:
        g = _sc_gather(table, idx_flat)

    nb = jnp.transpose(g.reshape(2 * c, b, n, _K), (1, 0, 2, 3))
    ctr = jnp.broadcast_to(feat[:, :, :, None], (b, 2 * c, n, _K))
    return jnp.concatenate(
        [nb[:, 0:c], ctr[:, 0:c], nb[:, c:], ctr[:, c:]], axis=1
    )


# speedup vs baseline: 1.1613x; 1.0002x over previous
import dataclasses
import functools

import jax
import jax.numpy as jnp
from jax.experimental import pallas as pl
from jax.experimental.pallas import tpu as pltpu
from jax.experimental.pallas import tpu_sc as plsc

_K = 20
_KPAD = 32
_R = 512

_INTERPRET = False


def _topk_body(x_ref, xt_ref, out_ref, dist_ref):
    b = pl.program_id(0)
    t = pl.program_id(1)
    n = x_ref.shape[2]

    x = x_ref[0]
    xt = xt_ref[0, pl.ds(t * _R, _R), :]
    x0, x1, x2 = x[0:1, :], x[1:2, :], x[2:3, :]
    a0, a1, a2 = xt[:, 0:1], xt[:, 1:2], xt[:, 2:3]
    xx = (x0 * x0 + x1 * x1) + x2 * x2
    rr = (a0 * a0 + a1 * a1) + a2 * a2
    s = jax.lax.dot_general(
        xt.astype(jnp.bfloat16), x.astype(jnp.bfloat16),
        (((1,), (0,)), ((), ())),
        preferred_element_type=jnp.float32,
    )
    inner = -2.0 * s
    dist_ref[...] = ((-xx) - inner) - rr

    col = jax.lax.broadcasted_iota(jnp.int32, (_R, n), 1)
    lane_k = jax.lax.broadcasted_iota(jnp.int32, (_R, _KPAD), 1)
    neg_inf = jnp.float32(-jnp.inf)

    def body(k, idx):
        d = dist_ref[...]
        m = jnp.max(d, axis=1, keepdims=True)
        cand = jnp.where(d == m, col, n)
        j = jnp.min(cand, axis=1, keepdims=True)
        dist_ref[...] = jnp.where(col == j, neg_inf, d)
        return jnp.where(lane_k == k, j, idx)

    idx = jax.lax.fori_loop(0, _K, body, jnp.zeros((_R, _KPAD), jnp.int32))
    del b
    out_ref[0] = idx


def _topk_indices(coor):
    b, c, n = coor.shape
    xt = jnp.transpose(coor, (0, 2, 1))
    return pl.pallas_call(
        _topk_body,
        grid=(b, n // _R),
        in_specs=[
            pl.BlockSpec((1, c, n), lambda bb, tt: (bb, 0, 0)),
            pl.BlockSpec((1, n, c), lambda bb, tt: (bb, 0, 0)),
        ],
        out_specs=pl.BlockSpec((1, _R, _KPAD), lambda bb, tt: (bb, tt, 0)),
        out_shape=jax.ShapeDtypeStruct((b, n, _KPAD), jnp.int32),
        scratch_shapes=[pltpu.VMEM((_R, n), jnp.float32)],
        compiler_params=pltpu.CompilerParams(
            dimension_semantics=("parallel", "parallel")),
        interpret=_INTERPRET,
    )(coor, xt)


_NW = 32
_CHUNK = 2048
_NCH = 6


def _sc_gather(table, idx_flat):
    m = idx_flat.shape[0]
    per_w = m // _NW
    w_per_b = m // table.shape[0] // per_w
    tab_w = table.shape[1]
    mesh = plsc.VectorSubcoreMesh(core_axis_name="c", subcore_axis_name="s")
    cp = pltpu.CompilerParams()
    if "needs_layout_passes" in pltpu.CompilerParams.__dataclass_fields__:
        cp = dataclasses.replace(cp, needs_layout_passes=False)

    @functools.partial(
        pl.kernel,
        out_type=jax.ShapeDtypeStruct((_NCH, m), jnp.float32),
        mesh=mesh,
        compiler_params=cp,
        scratch_types=[
            pltpu.VMEM((tab_w,), jnp.float32),
            pltpu.VMEM((_CHUNK,), jnp.int32),
            pltpu.VMEM((_NCH, _CHUNK), jnp.float32),
        ],
    )
    def gather_kernel(tab_hbm, i_hbm, o_hbm, tab_v, idx_v, out_v):
        wid = jax.lax.axis_index("s") * 2 + jax.lax.axis_index("c")
        base = wid * per_w
        pltpu.sync_copy(tab_hbm.at[wid // w_per_b], tab_v)

        @pl.loop(0, per_w, step=_CHUNK)
        def _(off):
            pltpu.sync_copy(i_hbm.at[pl.ds(base + off, _CHUNK)], idx_v)

            @pl.loop(0, _CHUNK, step=16)
            def _(g):
                fl = idx_v[pl.ds(g, 16)] * _NCH
                for ch in range(_NCH):
                    out_v[ch, pl.ds(g, 16)] = plsc.load_gather(
                        tab_v, [fl + ch])

            pltpu.sync_copy(out_v, o_hbm.at[:, pl.ds(base + off, _CHUNK)])

    return gather_kernel(table, idx_flat)


def kernel(coor, nor):
    b, c, n = coor.shape
    idx = _topk_indices(coor)
    idx_flat = idx[:, :, :_K].reshape(b * n * _K)

    feat = jnp.concatenate([coor, nor], axis=1)
    table = jnp.transpose(feat, (0, 2, 1)).reshape(b, n * 2 * c)

    if _INTERPRET:
        tab2 = table.reshape(b, n, 2 * c)
        g = jnp.take_along_axis(
            tab2, idx[:, :, :_K].reshape(b, n * _K)[:, :, None], axis=1
        )
        g = jnp.transpose(g.reshape(b * n * _K, 2 * c), (1, 0))
    else:
        g = _sc_gather(table, idx_flat)

    nb = jnp.transpose(g.reshape(2 * c, b, n, _K), (1, 0, 2, 3))
    ctr = jnp.broadcast_to(feat[:, :, :, None], (b, 2 * c, n, _K))
    return jnp.concatenate(
        [nb[:, 0:c], ctr[:, 0:c], nb[:, c:], ctr[:, c:]], axis=1
    )
